# Initial kernel scaffold; baseline (speedup 1.0000x reference)
import jax
import jax.numpy as jnp
from jax.experimental import pallas as pl


def kernel(anchor_embeddings, var_ids, edge_index, edge_type, mode_emb, comp0, basis0, root0, bias0, comp1, basis1, root1, bias1, comp2, basis2, root2, bias2):
    raise NotImplementedError("write your pallas kernel here")


# speedup vs baseline: 1.4548x; 1.4548x over previous
import functools

import jax
import jax.numpy as jnp
from jax import lax
from jax.experimental import pallas as pl
from jax.experimental.pallas import tpu as pltpu
from jax.experimental.pallas import tpu_sc as plsc

_NA = 3
_EMB = 64
_NR = 100
_CP = 128
_BT = 200


def _sc_gather_rows(table, idx, n_pad):
  info = plsc.get_sparse_core_info()
  nw = info.num_cores * info.num_subcores
  per = n_pad // nw
  d = table.shape[1]
  mesh = plsc.VectorSubcoreMesh(core_axis_name="c", subcore_axis_name="s")

  @functools.partial(
      pl.kernel,
      mesh=mesh,
      out_type=jax.ShapeDtypeStruct((n_pad, d), jnp.float32),
      scratch_types=[
          pltpu.VMEM((per,), jnp.int32),
          pltpu.VMEM((per, d), jnp.float32),
          pltpu.SemaphoreType.DMA,
      ],
  )
  def gather(table_hbm, idx_hbm, out_hbm, idx_v, rows_v, sem):
    wid = lax.axis_index("s") * info.num_cores + lax.axis_index("c")
    base = wid * per
    pltpu.sync_copy(idx_hbm.at[pl.ds(base, per)], idx_v)
    pltpu.async_copy(table_hbm.at[idx_v], rows_v, sem).wait()
    pltpu.sync_copy(rows_v, out_hbm.at[pl.ds(base, per)])

  return gather(table, idx)


def _rgcn_tc_body(anch_ref, m_ref, c0_ref, c1_ref, c2_ref,
                  bf0_ref, bf1_ref, bf2_ref,
                  r0_ref, r1_ref, r2_ref,
                  b0_ref, b1_ref, b2_ref, out_ref):
  a = [anch_ref[j] for j in range(_NA)]
  h = m_ref[...]
  c_refs = (c0_ref, c1_ref, c2_ref)
  bf_refs = (bf0_ref, bf1_ref, bf2_ref)
  r_refs = (r0_ref, r1_ref, r2_ref)
  b_refs = (b0_ref, b1_ref, b2_ref)
  for l in range(3):
    v = None
    for j in range(_NA):
      cj = c_refs[l][j, :, :_NR]
      t = cj[:, :, None] * a[j][:, None, :]
      v = t if v is None else v + t
    agg = jnp.dot(v.reshape(_BT, _NR * _EMB), bf_refs[l][...],
                  preferred_element_type=jnp.float32)
    rl = r_refs[l][...]
    bias = b_refs[l][...]
    h = agg + jnp.dot(h, rl, preferred_element_type=jnp.float32) + bias
    if l < 2:
      h = jnp.maximum(h, 0.0)
      a = [jnp.maximum(jnp.dot(a[j], rl, preferred_element_type=jnp.float32)
                       + bias, 0.0)
           for j in range(_NA)]
  out_ref[...] = h


def kernel(anchor_embeddings, var_ids, edge_index, edge_type, mode_emb,
           comp0, basis0, root0, bias0,
           comp1, basis1, root1, bias1,
           comp2, basis2, root2, bias2):
  del edge_index
  b = anchor_embeddings.shape[1]

  n_vid = ((b + 255) // 256) * 256
  vid = var_ids[:, 0].astype(jnp.int32)
  vid_pad = jnp.pad(vid, (0, n_vid - b))
  mode_p = jnp.pad(mode_emb, ((0, 0), (0, _CP - _EMB)))
  m = _sc_gather_rows(mode_p, vid_pad, n_vid)[:b, :_EMB]

  ne = 3 * b
  n_e = ((ne + 255) // 256) * 256
  t_jmaj = edge_type.astype(jnp.int32).reshape(b, _NA).T.reshape(-1)
  t_pad = jnp.pad(t_jmaj, (0, n_e - ne))
  cs = []
  for comp in (comp0, comp1, comp2):
    comp_p = jnp.pad(comp, ((0, 0), (0, _CP - _NR)))
    cs.append(_sc_gather_rows(comp_p, t_pad, n_e)[:ne].reshape(_NA, b, _CP))

  bfs = [x.reshape(_NR * _EMB, _EMB) for x in (basis0, basis1, basis2)]
  biases = [x.reshape(1, _EMB) for x in (bias0, bias1, bias2)]
  wspec = lambda shape: pl.BlockSpec(shape, lambda g: tuple(0 for _ in shape))
  out = pl.pallas_call(
      _rgcn_tc_body,
      grid=(b // _BT,),
      in_specs=[
          pl.BlockSpec((_NA, _BT, _EMB), lambda g: (0, g, 0)),
          pl.BlockSpec((_BT, _EMB), lambda g: (g, 0)),
          pl.BlockSpec((_NA, _BT, _CP), lambda g: (0, g, 0)),
          pl.BlockSpec((_NA, _BT, _CP), lambda g: (0, g, 0)),
          pl.BlockSpec((_NA, _BT, _CP), lambda g: (0, g, 0)),
          wspec((_NR * _EMB, _EMB)),
          wspec((_NR * _EMB, _EMB)),
          wspec((_NR * _EMB, _EMB)),
          wspec((_EMB, _EMB)),
          wspec((_EMB, _EMB)),
          wspec((_EMB, _EMB)),
          wspec((1, _EMB)),
          wspec((1, _EMB)),
          wspec((1, _EMB)),
      ],
      out_specs=pl.BlockSpec((_BT, _EMB), lambda g: (g, 0)),
      out_shape=jax.ShapeDtypeStruct((b, _EMB), jnp.float32),
  )(anchor_embeddings, m, cs[0], cs[1], cs[2],
    bfs[0], bfs[1], bfs[2], root0, root1, root2,
    biases[0], biases[1], biases[2])
  return out


# speedup vs baseline: 1.4930x; 1.0263x over previous
import functools

import jax
import jax.numpy as jnp
from jax import lax
from jax.experimental import pallas as pl
from jax.experimental.pallas import tpu as pltpu
from jax.experimental.pallas import tpu_sc as plsc

_NA = 3
_EMB = 64
_NR = 100
_CP = 128
_BT = 200
_CHUNKS = 2


def _sc_gather_rows(table, idx, n_pad):
  info = plsc.get_sparse_core_info()
  nw = info.num_cores * info.num_subcores
  chunk = n_pad // (nw * _CHUNKS)
  mesh = plsc.VectorSubcoreMesh(core_axis_name="c", subcore_axis_name="s")

  @functools.partial(
      pl.kernel,
      mesh=mesh,
      out_type=jax.ShapeDtypeStruct((n_pad, _CP), jnp.float32),
      scratch_types=[
          pltpu.VMEM((chunk,), jnp.int32),
          pltpu.VMEM((chunk, _CP), jnp.float32),
          pltpu.SemaphoreType.DMA,
      ],
  )
  def gather(table_hbm, idx_hbm, out_hbm, idx_v, rows_v, sem):
    wid = lax.axis_index("s") * info.num_cores + lax.axis_index("c")
    for c in range(_CHUNKS):
      base = (wid * _CHUNKS + c) * chunk
      pltpu.sync_copy(idx_hbm.at[pl.ds(base, chunk)], idx_v)
      pltpu.async_copy(table_hbm.at[idx_v], rows_v, sem).wait()
      pltpu.sync_copy(rows_v, out_hbm.at[pl.ds(base, chunk)])

  return gather(table, idx)


def _rgcn_tc_body(anch_ref, m_ref,
                  c00, c01, c02, c10, c11, c12, c20, c21, c22,
                  bf0_ref, bf1_ref, bf2_ref,
                  r0_ref, r1_ref, r2_ref,
                  b0_ref, b1_ref, b2_ref, out_ref):
  a = [anch_ref[j] for j in range(_NA)]
  h = m_ref[:, :_EMB]
  c_refs = ((c00, c01, c02), (c10, c11, c12), (c20, c21, c22))
  bf_refs = (bf0_ref, bf1_ref, bf2_ref)
  r_refs = (r0_ref, r1_ref, r2_ref)
  b_refs = (b0_ref, b1_ref, b2_ref)
  for l in range(3):
    v = None
    for j in range(_NA):
      cj = c_refs[l][j][:, :_NR]
      t = cj[:, :, None] * a[j][:, None, :]
      v = t if v is None else v + t
    agg = jnp.dot(v.reshape(_BT, _NR * _EMB), bf_refs[l][...],
                  preferred_element_type=jnp.float32)
    rl = r_refs[l][...]
    bias = b_refs[l][...]
    h = agg + jnp.dot(h, rl, preferred_element_type=jnp.float32) + bias
    if l < 2:
      h = jnp.maximum(h, 0.0)
      a = [jnp.maximum(jnp.dot(a[j], rl, preferred_element_type=jnp.float32)
                       + bias, 0.0)
           for j in range(_NA)]
  out_ref[...] = h


def kernel(anchor_embeddings, var_ids, edge_index, edge_type, mode_emb,
           comp0, basis0, root0, bias0,
           comp1, basis1, root1, bias1,
           comp2, basis2, root2, bias2):
  del edge_index
  b = anchor_embeddings.shape[1]
  nm = mode_emb.shape[0]
  ne = _NA * b

  table = jnp.concatenate([
      jnp.pad(mode_emb, ((0, 0), (0, _CP - _EMB))),
      jnp.pad(comp0, ((0, 0), (0, _CP - _NR))),
      jnp.pad(comp1, ((0, 0), (0, _CP - _NR))),
      jnp.pad(comp2, ((0, 0), (0, _CP - _NR))),
  ], axis=0)
  t_jmaj = edge_type.astype(jnp.int32).reshape(b, _NA).T.reshape(-1)
  n_rows = b + 3 * ne
  n_pad = ((n_rows + 511) // 512) * 512
  idx = jnp.concatenate([
      var_ids[:, 0].astype(jnp.int32),
      t_jmaj + nm,
      t_jmaj + nm + _NR,
      t_jmaj + nm + 2 * _NR,
      jnp.zeros((n_pad - n_rows,), jnp.int32),
  ])
  rows = _sc_gather_rows(table, idx, n_pad)

  def cmap(l, j):
    off = (b + l * ne + j * b) // _BT
    return lambda g: (off + g, 0)

  bfs = [x.reshape(_NR * _EMB, _EMB) for x in (basis0, basis1, basis2)]
  biases = [x.reshape(1, _EMB) for x in (bias0, bias1, bias2)]
  wspec = lambda shape: pl.BlockSpec(shape, lambda g: tuple(0 for _ in shape))
  cspecs = [pl.BlockSpec((_BT, _CP), cmap(l, j))
            for l in range(3) for j in range(_NA)]
  out = pl.pallas_call(
      _rgcn_tc_body,
      grid=(b // _BT,),
      in_specs=[
          pl.BlockSpec((_NA, _BT, _EMB), lambda g: (0, g, 0)),
          pl.BlockSpec((_BT, _CP), lambda g: (g, 0)),
          *cspecs,
          wspec((_NR * _EMB, _EMB)),
          wspec((_NR * _EMB, _EMB)),
          wspec((_NR * _EMB, _EMB)),
          wspec((_EMB, _EMB)),
          wspec((_EMB, _EMB)),
          wspec((_EMB, _EMB)),
          wspec((1, _EMB)),
          wspec((1, _EMB)),
          wspec((1, _EMB)),
      ],
      out_specs=pl.BlockSpec((_BT, _EMB), lambda g: (g, 0)),
      out_shape=jax.ShapeDtypeStruct((b, _EMB), jnp.float32),
  )(anchor_embeddings, rows, *([rows] * 9),
    bfs[0], bfs[1], bfs[2], root0, root1, root2,
    biases[0], biases[1], biases[2])
  return out


# speedup vs baseline: 6.2139x; 4.1620x over previous
import functools

import jax
import jax.numpy as jnp
from jax import lax
from jax.experimental import pallas as pl
from jax.experimental.pallas import tpu as pltpu
from jax.experimental.pallas import tpu_sc as plsc

_NA = 3
_EMB = 64
_NR = 100
_CP = 128
_BT = 256
_BP = 4096
_CHUNKS = 2


def _sc_gather_rows(table, idx, n_pad):
  info = plsc.get_sparse_core_info()
  nw = info.num_cores * info.num_subcores
  chunk = n_pad // (nw * _CHUNKS)
  mesh = plsc.VectorSubcoreMesh(core_axis_name="c", subcore_axis_name="s")

  @functools.partial(
      pl.kernel,
      mesh=mesh,
      out_type=jax.ShapeDtypeStruct((n_pad, _CP), jnp.float32),
      scratch_types=[
          pltpu.VMEM((chunk,), jnp.int32),
          pltpu.VMEM((chunk, _CP), jnp.float32),
          pltpu.SemaphoreType.DMA,
      ],
  )
  def gather(table_hbm, idx_hbm, out_hbm, idx_v, rows_v, sem):
    wid = lax.axis_index("s") * info.num_cores + lax.axis_index("c")
    for c in range(_CHUNKS):
      base = (wid * _CHUNKS + c) * chunk
      pltpu.sync_copy(idx_hbm.at[pl.ds(base, chunk)], idx_v)
      pltpu.async_copy(table_hbm.at[idx_v], rows_v, sem).wait()
      pltpu.sync_copy(rows_v, out_hbm.at[pl.ds(base, chunk)])

  return gather(table, idx)


def _rgcn_tc_body(anch_ref, m_ref,
                  c00, c01, c02, c10, c11, c12, c20, c21, c22,
                  bf0_ref, bf1_ref, bf2_ref,
                  r0_ref, r1_ref, r2_ref,
                  b0_ref, b1_ref, b2_ref, out_ref):
  a = [anch_ref[j] for j in range(_NA)]
  h = jnp.transpose(m_ref[...])[:_EMB]
  c_refs = ((c00, c01, c02), (c10, c11, c12), (c20, c21, c22))
  bf_refs = (bf0_ref, bf1_ref, bf2_ref)
  r_refs = (r0_ref, r1_ref, r2_ref)
  b_refs = (b0_ref, b1_ref, b2_ref)
  for l in range(3):
    v = None
    for j in range(_NA):
      cj = jnp.transpose(c_refs[l][j][...])[:_NR]
      t = cj[:, None, :] * a[j][None, :, :]
      v = t if v is None else v + t
    agg = jnp.dot(bf_refs[l][...], v.reshape(_NR * _EMB, _BT),
                  preferred_element_type=jnp.float32)
    rl = r_refs[l][...]
    bias = b_refs[l][...]
    h = agg + jnp.dot(rl, h, preferred_element_type=jnp.float32) + bias
    if l < 2:
      h = jnp.maximum(h, 0.0)
      a = [jnp.maximum(jnp.dot(rl, a[j], preferred_element_type=jnp.float32)
                       + bias, 0.0)
           for j in range(_NA)]
  out_ref[...] = h


def kernel(anchor_embeddings, var_ids, edge_index, edge_type, mode_emb,
           comp0, basis0, root0, bias0,
           comp1, basis1, root1, bias1,
           comp2, basis2, root2, bias2):
  del edge_index
  b = anchor_embeddings.shape[1]
  nm = mode_emb.shape[0]

  table = jnp.concatenate([
      jnp.pad(mode_emb, ((0, 0), (0, _CP - _EMB))),
      jnp.pad(comp0, ((0, 0), (0, _CP - _NR))),
      jnp.pad(comp1, ((0, 0), (0, _CP - _NR))),
      jnp.pad(comp2, ((0, 0), (0, _CP - _NR))),
  ], axis=0)
  tj = jnp.pad(edge_type.astype(jnp.int32).reshape(b, _NA).T,
               ((0, 0), (0, _BP - b)))
  vid = jnp.pad(var_ids[:, 0].astype(jnp.int32), (0, _BP - b))
  segs = [vid] + [tj[j] + nm + l * _NR
                  for l in range(3) for j in range(_NA)]
  n_pad = 10 * _BP
  idx = jnp.concatenate(segs)
  rows = _sc_gather_rows(table, idx, n_pad)

  def cmap(l, j):
    off = (1 + 3 * l + j) * (_BP // _BT)
    return lambda g: (off + g, 0)

  anch_t = jnp.pad(anchor_embeddings.transpose(0, 2, 1),
                   ((0, 0), (0, 0), (0, _BP - b)))
  bfs = [x.transpose(2, 0, 1).reshape(_EMB, _NR * _EMB)
         for x in (basis0, basis1, basis2)]
  roots_t = [x.T for x in (root0, root1, root2)]
  biases = [x.reshape(_EMB, 1) for x in (bias0, bias1, bias2)]
  wspec = lambda shape: pl.BlockSpec(shape, lambda g: tuple(0 for _ in shape))
  cspecs = [pl.BlockSpec((_BT, _CP), cmap(l, j))
            for l in range(3) for j in range(_NA)]
  out = pl.pallas_call(
      _rgcn_tc_body,
      grid=(_BP // _BT,),
      in_specs=[
          pl.BlockSpec((_NA, _EMB, _BT), lambda g: (0, 0, g)),
          pl.BlockSpec((_BT, _CP), lambda g: (g, 0)),
          *cspecs,
          wspec((_EMB, _NR * _EMB)),
          wspec((_EMB, _NR * _EMB)),
          wspec((_EMB, _NR * _EMB)),
          wspec((_EMB, _EMB)),
          wspec((_EMB, _EMB)),
          wspec((_EMB, _EMB)),
          wspec((_EMB, 1)),
          wspec((_EMB, 1)),
          wspec((_EMB, 1)),
      ],
      out_specs=pl.BlockSpec((_EMB, _BT), lambda g: (0, g)),
      out_shape=jax.ShapeDtypeStruct((_EMB, _BP), jnp.float32),
  )(anch_t, rows, *([rows] * 9),
    bfs[0], bfs[1], bfs[2], roots_t[0], roots_t[1], roots_t[2],
    biases[0], biases[1], biases[2])
  return out[:, :b].T
